# baseline (speedup 1.0000x reference)
import functools

import jax
import jax.numpy as jnp
from jax.experimental import pallas as pl
from jax.experimental.pallas import tpu as pltpu

_N_PREDICT = 3
_H = 224
_W = 224
_R = _N_PREDICT * _H
_F = _R * _W


def _make_body(B):
    def body(vec_ref, out_hbm, scratch, sems):
        for r in range(_R):
            scratch[0, r, :] = vec_ref[0, pl.ds(r * _W, _W)]
        scratch[1] = scratch[0]
        scratch[2:4] = scratch[0:2]
        copies = [
            pltpu.async_copy(
                scratch, out_hbm.at[pl.ds(4 * i, 4)], sems.at[i]
            )
            for i in range(B // 4)
        ]
        for c in copies:
            c.wait()

    return body


@functools.partial(jax.jit, static_argnums=(1,))
def _tc_broadcast(vec, B):
    out = pl.pallas_call(
        _make_body(B),
        in_specs=[pl.BlockSpec((1, _F), lambda: (0, 0))],
        out_specs=pl.BlockSpec(memory_space=pl.ANY),
        out_shape=jax.ShapeDtypeStruct((B, _R, _W), jnp.float32),
        scratch_shapes=[
            pltpu.VMEM((4, _R, _W), jnp.float32),
            pltpu.SemaphoreType.DMA((4,)),
        ],
    )(vec)
    return out.reshape(B, _N_PREDICT, _H, _W)


def kernel(era5_land, guess):
    B = era5_land.shape[0]
    return _tc_broadcast(guess, B)


# speedup vs baseline: 1.0777x; 1.0777x over previous
import functools

import jax
import jax.numpy as jnp
from jax.experimental import pallas as pl
from jax.experimental.pallas import tpu as pltpu

_N_PREDICT = 3
_H = 224
_W = 224
_R = _N_PREDICT * _H
_F = _R * _W


def _make_body(B):
    def body(vec_ref, out_hbm, scratch, sems):
        half = _R // 2
        copies = []
        for h in range(2):
            for r in range(h * half, (h + 1) * half):
                scratch[r, :] = vec_ref[0, pl.ds(r * _W, _W)]
            copies += [
                pltpu.async_copy(
                    scratch.at[pl.ds(h * half, half)],
                    out_hbm.at[b, pl.ds(h * half, half)],
                    sems.at[(2 * b + h) % 4],
                )
                for b in range(B)
            ]
        for c in copies:
            c.wait()

    return body


@functools.partial(jax.jit, static_argnums=(1,))
def _tc_broadcast(vec, B):
    out = pl.pallas_call(
        _make_body(B),
        in_specs=[pl.BlockSpec((1, _F), lambda: (0, 0))],
        out_specs=pl.BlockSpec(memory_space=pl.ANY),
        out_shape=jax.ShapeDtypeStruct((B, _R, _W), jnp.float32),
        scratch_shapes=[
            pltpu.VMEM((_R, _W), jnp.float32),
            pltpu.SemaphoreType.DMA((4,)),
        ],
    )(vec)
    return out.reshape(B, _N_PREDICT, _H, _W)


def kernel(era5_land, guess):
    B = era5_land.shape[0]
    return _tc_broadcast(guess, B)


# speedup vs baseline: 1.1045x; 1.0249x over previous
import functools

import jax
import jax.numpy as jnp
from jax.experimental import pallas as pl
from jax.experimental.pallas import tpu as pltpu

_N_PREDICT = 3
_H = 224
_W = 224
_R = _N_PREDICT * _H
_F = _R * _W


def _make_body(B):
    def body(vec_ref, out_hbm, scratch, sems):
        nch = 4
        chunk = _R // nch
        copies = []
        for h in range(nch):
            for r in range(h * chunk, (h + 1) * chunk):
                scratch[r, :] = vec_ref[0, pl.ds(r * _W, _W)]
            copies += [
                pltpu.async_copy(
                    scratch.at[pl.ds(h * chunk, chunk)],
                    out_hbm.at[b, pl.ds(h * chunk, chunk)],
                    sems.at[(nch * b + h) % 4],
                )
                for b in range(B)
            ]
        for c in copies:
            c.wait()

    return body


@functools.partial(jax.jit, static_argnums=(1,))
def _tc_broadcast(vec, B):
    out = pl.pallas_call(
        _make_body(B),
        in_specs=[pl.BlockSpec((1, _F), lambda: (0, 0))],
        out_specs=pl.BlockSpec(memory_space=pl.ANY),
        out_shape=jax.ShapeDtypeStruct((B, _R, _W), jnp.float32),
        scratch_shapes=[
            pltpu.VMEM((_R, _W), jnp.float32),
            pltpu.SemaphoreType.DMA((4,)),
        ],
    )(vec)
    return out.reshape(B, _N_PREDICT, _H, _W)


def kernel(era5_land, guess):
    B = era5_land.shape[0]
    return _tc_broadcast(guess, B)


# speedup vs baseline: 1.1179x; 1.0122x over previous
import functools

import jax
import jax.numpy as jnp
from jax.experimental import pallas as pl
from jax.experimental.pallas import tpu as pltpu

_N_PREDICT = 3
_H = 224
_W = 224
_R = _N_PREDICT * _H
_F = _R * _W


def _make_body(B):
    def body(vec_hbm, out_hbm, vbuf, scratch, sems, insems):
        nch = 4
        chunk = _R // nch
        cf = chunk * _W
        fetches = [
            pltpu.async_copy(
                vec_hbm.at[:, pl.ds(h * cf, cf)],
                vbuf.at[:, pl.ds(h * cf, cf)],
                insems.at[h],
            )
            for h in range(nch)
        ]
        copies = []
        for h in range(nch):
            fetches[h].wait()
            for r in range(h * chunk, (h + 1) * chunk):
                scratch[r, :] = vbuf[0, pl.ds(r * _W, _W)]
            copies += [
                pltpu.async_copy(
                    scratch.at[pl.ds(h * chunk, chunk)],
                    out_hbm.at[b, pl.ds(h * chunk, chunk)],
                    sems.at[(nch * b + h) % 4],
                )
                for b in range(B)
            ]
        for c in copies:
            c.wait()

    return body


@functools.partial(jax.jit, static_argnums=(1,))
def _tc_broadcast(vec, B):
    out = pl.pallas_call(
        _make_body(B),
        in_specs=[pl.BlockSpec(memory_space=pl.ANY)],
        out_specs=pl.BlockSpec(memory_space=pl.ANY),
        out_shape=jax.ShapeDtypeStruct((B, _R, _W), jnp.float32),
        scratch_shapes=[
            pltpu.VMEM((1, _F), jnp.float32),
            pltpu.VMEM((_R, _W), jnp.float32),
            pltpu.SemaphoreType.DMA((4,)),
            pltpu.SemaphoreType.DMA((4,)),
        ],
    )(vec)
    return out.reshape(B, _N_PREDICT, _H, _W)


def kernel(era5_land, guess):
    B = era5_land.shape[0]
    return _tc_broadcast(guess, B)


# speedup vs baseline: 1.1449x; 1.0242x over previous
import functools

import jax
import jax.numpy as jnp
from jax.experimental import pallas as pl
from jax.experimental.pallas import tpu as pltpu

_N_PREDICT = 3
_H = 224
_W = 224
_R = _N_PREDICT * _H
_F = _R * _W


def _make_body(B):
    def body(vec_hbm, out_hbm, vbuf, scratch, sems, insems):
        nch = 7
        chunk = _R // nch
        cf = chunk * _W
        fetches = [
            pltpu.async_copy(
                vec_hbm.at[:, pl.ds(h * cf, cf)],
                vbuf.at[:, pl.ds(h * cf, cf)],
                insems.at[h],
            )
            for h in range(nch)
        ]
        copies = []
        for h in range(nch):
            fetches[h].wait()
            for r in range(h * chunk, (h + 1) * chunk):
                scratch[r, :] = vbuf[0, pl.ds(r * _W, _W)]
            copies += [
                pltpu.async_copy(
                    scratch.at[pl.ds(h * chunk, chunk)],
                    out_hbm.at[b, pl.ds(h * chunk, chunk)],
                    sems.at[(nch * b + h) % 4],
                )
                for b in range(B)
            ]
        for c in copies:
            c.wait()

    return body


@functools.partial(jax.jit, static_argnums=(1,))
def _tc_broadcast(vec, B):
    out = pl.pallas_call(
        _make_body(B),
        in_specs=[pl.BlockSpec(memory_space=pl.ANY)],
        out_specs=pl.BlockSpec(memory_space=pl.ANY),
        out_shape=jax.ShapeDtypeStruct((B, _R, _W), jnp.float32),
        scratch_shapes=[
            pltpu.VMEM((1, _F), jnp.float32),
            pltpu.VMEM((_R, _W), jnp.float32),
            pltpu.SemaphoreType.DMA((4,)),
            pltpu.SemaphoreType.DMA((7,)),
        ],
    )(vec)
    return out.reshape(B, _N_PREDICT, _H, _W)


def kernel(era5_land, guess):
    B = era5_land.shape[0]
    return _tc_broadcast(guess, B)


# speedup vs baseline: 1.1510x; 1.0053x over previous
import functools

import jax
import jax.numpy as jnp
from jax.experimental import pallas as pl
from jax.experimental.pallas import tpu as pltpu

_N_PREDICT = 3
_H = 224
_W = 224
_R = _N_PREDICT * _H
_F = _R * _W


def _make_body(B):
    def body(vec_hbm, out_hbm, vbuf, scratch, sems, insems):
        nch = 12
        chunk = _R // nch
        cf = chunk * _W
        fetches = [
            pltpu.async_copy(
                vec_hbm.at[:, pl.ds(h * cf, cf)],
                vbuf.at[:, pl.ds(h * cf, cf)],
                insems.at[h],
            )
            for h in range(nch)
        ]
        copies = []
        for h in range(nch):
            fetches[h].wait()
            for r in range(h * chunk, (h + 1) * chunk):
                scratch[r, :] = vbuf[0, pl.ds(r * _W, _W)]
            copies += [
                pltpu.async_copy(
                    scratch.at[pl.ds(h * chunk, chunk)],
                    out_hbm.at[b, pl.ds(h * chunk, chunk)],
                    sems.at[(nch * b + h) % 4],
                )
                for b in range(B)
            ]
        for c in copies:
            c.wait()

    return body


@functools.partial(jax.jit, static_argnums=(1,))
def _tc_broadcast(vec, B):
    out = pl.pallas_call(
        _make_body(B),
        in_specs=[pl.BlockSpec(memory_space=pl.ANY)],
        out_specs=pl.BlockSpec(memory_space=pl.ANY),
        out_shape=jax.ShapeDtypeStruct((B, _R, _W), jnp.float32),
        scratch_shapes=[
            pltpu.VMEM((1, _F), jnp.float32),
            pltpu.VMEM((_R, _W), jnp.float32),
            pltpu.SemaphoreType.DMA((4,)),
            pltpu.SemaphoreType.DMA((12,)),
        ],
    )(vec)
    return out.reshape(B, _N_PREDICT, _H, _W)


def kernel(era5_land, guess):
    B = era5_land.shape[0]
    return _tc_broadcast(guess, B)
